# Initial kernel scaffold; baseline (speedup 1.0000x reference)
import jax
import jax.numpy as jnp
from jax.experimental import pallas as pl


def kernel(x, w1a1, w1a2, b1a, w1b1, w1b2, b1b, w21, w22, b2, edge_index):
    raise NotImplementedError("write your pallas kernel here")


# speedup vs baseline: 15.9985x; 15.9985x over previous
import functools

import jax
import jax.numpy as jnp
from jax import lax
from jax.experimental import pallas as pl
from jax.experimental.pallas import tpu as pltpu
from jax.experimental.pallas import tpu_sc as plsc

_N = 10000
_NC = 2
_NS = 16
_NW = _NC * _NS
_IDX = 128
_ROWS = 10240


def _elu(v):
    return jnp.where(v > 0.0, v, jnp.exp(jnp.minimum(v, 0.0)) - 1.0)


def _tc1_body(x_ref, wg_ref, ws_ref, g_ref, s_ref):
    xv = x_ref[...]
    g_ref[...] = jnp.dot(xv, wg_ref[...], preferred_element_type=jnp.float32)
    s_ref[...] = jnp.dot(xv, ws_ref[...], preferred_element_type=jnp.float32)


def _tc2_body(a_ref, s_ref, b1a_ref, b1b_ref, h_ref):
    a = a_ref[0] + a_ref[1]
    a = a[:_N]
    s = s_ref[...]
    ha = _elu(a[:, :16] + s[:, :16] + b1a_ref[...])
    hb = _elu(a[:, 16:] + s[:, 16:] + b1b_ref[...])
    h_ref[...] = _elu(0.5 * (ha + hb))


def _tc3_body(a_ref, h_ref, w21_ref, w22_ref, b2_ref, o_ref):
    a = a_ref[0] + a_ref[1]
    a = a[:_N]
    h = h_ref[...]
    z = (jnp.dot(a, w21_ref[...], preferred_element_type=jnp.float32)
         + jnp.dot(h, w22_ref[...], preferred_element_type=jnp.float32)
         + b2_ref[...])
    z = z - jnp.max(z, axis=-1, keepdims=True)
    e = jnp.exp(z)
    o_ref[...] = e / jnp.sum(e, axis=-1, keepdims=True)


def _make_sc_agg(d, n_chunks):
    rpt = _ROWS // _NS

    def body(g_hbm, src_hbm, dst_hbm, z_hbm, out_hbm,
             src_v, dst_v, rows_v, acc_sh, sem):
        c = lax.axis_index("c")
        s = lax.axis_index("s")
        wid = s * _NC + c
        pltpu.sync_copy(z_hbm.at[pl.ds(s * rpt, rpt)],
                        acc_sh.at[pl.ds(s * rpt, rpt)])
        pltpu.sync_copy(src_hbm.at[wid], src_v)
        pltpu.sync_copy(dst_hbm.at[wid], dst_v)
        plsc.subcore_barrier()

        def step(j, carry):
            pltpu.async_copy(g_hbm.at[src_v.at[j]], rows_v, sem).wait()
            pltpu.sync_copy(rows_v, acc_sh.at[dst_v.at[j]], add=True)
            return carry

        lax.fori_loop(0, n_chunks, step, 0)
        plsc.subcore_barrier()
        pltpu.sync_copy(acc_sh.at[pl.ds(s * rpt, rpt)],
                        out_hbm.at[c, pl.ds(s * rpt, rpt)])

    return pl.kernel(
        body,
        mesh=plsc.VectorSubcoreMesh(core_axis_name="c", subcore_axis_name="s"),
        compiler_params=pltpu.CompilerParams(use_tc_tiling_on_sc=False),
        out_type=jax.ShapeDtypeStruct((_NC, _ROWS, d), jnp.float32),
        scratch_types=[
            pltpu.VMEM((n_chunks, _IDX), jnp.int32),
            pltpu.VMEM((n_chunks, _IDX), jnp.int32),
            pltpu.VMEM((_IDX, d), jnp.float32),
            pltpu.VMEM_SHARED((_ROWS, d), jnp.float32),
            pltpu.SemaphoreType.DMA,
        ],
    )


@jax.jit
def kernel(x, w1a1, w1a2, b1a, w1b1, w1b2, b1b, w21, w22, b2, edge_index):
    n, f_in = x.shape
    h1 = w1a1.shape[1]
    h2 = w21.shape[1]
    e = edge_index.shape[1]

    epw = _NW * _IDX
    ep = ((e + epw - 1) // epw) * epw
    n_chunks = ep // epw
    src = jnp.concatenate(
        [edge_index[0], jnp.zeros((ep - e,), dtype=jnp.int32)])
    dst = jnp.concatenate(
        [edge_index[1], jnp.full((ep - e,), _N, dtype=jnp.int32)])
    src3 = src.reshape(_NW, n_chunks, _IDX)
    dst3 = dst.reshape(_NW, n_chunks, _IDX)

    pad = jnp.zeros((f_in - w1a1.shape[0], 2 * h1), dtype=jnp.float32)
    wg = jnp.concatenate(
        [jnp.concatenate([w1a1, w1b1], axis=1), pad], axis=0)
    ws = jnp.concatenate(
        [jnp.concatenate([w1a2, w1b2], axis=1), pad], axis=0)

    g1, s1 = pl.pallas_call(
        _tc1_body,
        out_shape=[jax.ShapeDtypeStruct((n, 2 * h1), jnp.float32),
                   jax.ShapeDtypeStruct((n, 2 * h1), jnp.float32)],
    )(x, wg, ws)

    z32 = jnp.zeros((_ROWS, 2 * h1), dtype=jnp.float32)
    a1 = _make_sc_agg(2 * h1, n_chunks)(g1, src3, dst3, z32)

    h = pl.pallas_call(
        _tc2_body,
        out_shape=jax.ShapeDtypeStruct((n, h1), jnp.float32),
    )(a1, s1, b1a.reshape(1, h1), b1b.reshape(1, h1))

    z16 = jnp.zeros((_ROWS, h1), dtype=jnp.float32)
    a2 = _make_sc_agg(h1, n_chunks)(h, src3, dst3, z16)

    return pl.pallas_call(
        _tc3_body,
        out_shape=jax.ShapeDtypeStruct((n, h2), jnp.float32),
    )(a2, h, w21, w22, b2.reshape(1, h2))
